# baseline (speedup 1.0000x reference)
import jax
import jax.numpy as jnp
from jax import lax
from jax.experimental import pallas as pl
from jax.experimental.pallas import tpu as pltpu
from jax.experimental.pallas import tpu_sc as plsc

NC = 2
NS = 16
L = 16
NW = NC * NS
D = 128
NZ = 10000
E = 320000
EPW = 2 * E // NW
CHUNK = 48
NCHP = -(-EPW // CHUNK) + 1
PADE = NCHP * CHUNK
NPAIR = NCHP // 2
GRPS = CHUNK // L
KV = D // L
EPS = 1e-15
LN2 = 0.6931471805599453


def _vlog(x):
    bits = plsc.bitcast(x, jnp.int32)
    e = (bits >> 23) - 127
    m = plsc.bitcast((bits & 0x7FFFFF) | 0x3F800000, jnp.float32)
    big = m >= jnp.float32(1.4142135)
    m = jnp.where(big, m * jnp.float32(0.5), m)
    ef = (e + big.astype(jnp.int32)).astype(jnp.float32)
    s = (m - jnp.float32(1.0)) / (m + jnp.float32(1.0))
    u = s * s
    p = jnp.float32(1.0 / 11.0)
    for c in (1.0 / 9.0, 1.0 / 7.0, 1.0 / 5.0, 1.0 / 3.0):
        p = p * u + jnp.float32(c)
    return ef * jnp.float32(LN2) + jnp.float32(2.0) * s * (jnp.float32(1.0) + u * p)


def _body(z_hbm, src_hbm, dst_hbm, out_hbm,
          idx_s0, idx_d0, idx_s1, idx_d1,
          rows_s0, rows_d0, rows_s1, rows_d1,
          scr, accv, redv, outv, shared, z_sh,
          sem_i0, sem_i1, sem_r0, sem_r1):
    cid = lax.axis_index("c")
    sid = lax.axis_index("s")
    wid = sid * NC + cid
    base_w = wid * EPW
    negv = jnp.full((L,), wid >= NW // 2)
    iota = lax.iota(jnp.int32, L)

    @pl.when(sid < NS - 1)
    def _():
        pltpu.sync_copy(z_hbm.at[pl.ds(sid * 624, 624)],
                        z_sh.at[pl.ds(sid * 624, 624)])

    @pl.when(sid == NS - 1)
    def _():
        pltpu.sync_copy(z_hbm.at[pl.ds(9360, 640)],
                        z_sh.at[pl.ds(9360, 640)])

    def fire_idx(c, idx_s, idx_d, sem):
        base = base_w + c * CHUNK
        pltpu.async_copy(src_hbm.at[pl.ds(base, CHUNK)], idx_s, sem)
        pltpu.async_copy(dst_hbm.at[pl.ds(base, CHUNK)], idx_d, sem)

    def drain_idx(idx_s, idx_d, sem):
        pltpu.make_async_copy(src_hbm.at[pl.ds(0, CHUNK)], idx_s, sem).wait()
        pltpu.make_async_copy(src_hbm.at[pl.ds(0, CHUNK)], idx_d, sem).wait()

    def fire_rows(idx_s, idx_d, rows_s, rows_d, sem):
        pltpu.async_copy(z_sh.at[idx_s], rows_s, sem)
        pltpu.async_copy(z_sh.at[idx_d], rows_d, sem)

    def drain_rows(rows_s, rows_d, sem):
        pltpu.make_async_copy(z_hbm.at[pl.ds(0, CHUNK)], rows_s, sem).wait()
        pltpu.make_async_copy(z_hbm.at[pl.ds(0, CHUNK)], rows_d, sem).wait()

    def compute(c, rows_s, rows_d, acc):
        for g in range(GRPS):
            sbase = g * (L * L)
            for e_ in range(L):
                r = g * L + e_
                a = rows_s[r, pl.ds(0, L)] * rows_d[r, pl.ds(0, L)]
                for k in range(1, KV):
                    a = a + rows_s[r, pl.ds(k * L, L)] * rows_d[r, pl.ds(k * L, L)]
                scr[pl.ds(sbase + e_ * L, L)] = a
            t = plsc.load_gather(scr, [iota * L + sbase])
            for l in range(1, L):
                t = t + plsc.load_gather(scr, [iota * L + (sbase + l)])
            prob = jnp.float32(1.0) / (jnp.float32(1.0) + jnp.exp(-t))
            arg = jnp.where(negv,
                            (jnp.float32(1.0) - prob) + jnp.float32(EPS),
                            prob + jnp.float32(EPS))
            gid = c * CHUNK + (g * L) + iota
            acc = acc - jnp.where(gid < EPW, _vlog(arg), jnp.float32(0.0))
        return acc

    plsc.subcore_barrier()

    pltpu.sync_copy(src_hbm.at[pl.ds(base_w, CHUNK)], idx_s0)
    pltpu.sync_copy(dst_hbm.at[pl.ds(base_w, CHUNK)], idx_d0)
    fire_rows(idx_s0, idx_d0, rows_s0, rows_d0, sem_r0)
    fire_idx(1, idx_s1, idx_d1, sem_i1)

    def pair_body(i, acc):
        drain_idx(idx_s1, idx_d1, sem_i1)
        fire_rows(idx_s1, idx_d1, rows_s1, rows_d1, sem_r1)

        @pl.when(i < NPAIR - 1)
        def _():
            fire_idx(2 * i + 2, idx_s0, idx_d0, sem_i0)

        drain_rows(rows_s0, rows_d0, sem_r0)
        acc = compute(2 * i, rows_s0, rows_d0, acc)

        @pl.when(i < NPAIR - 1)
        def _():
            drain_idx(idx_s0, idx_d0, sem_i0)
            fire_rows(idx_s0, idx_d0, rows_s0, rows_d0, sem_r0)
            fire_idx(2 * i + 3, idx_s1, idx_d1, sem_i1)

        drain_rows(rows_s1, rows_d1, sem_r1)
        acc = compute(2 * i + 1, rows_s1, rows_d1, acc)
        return acc

    acc = lax.fori_loop(0, NPAIR, pair_body, jnp.zeros((L,), jnp.float32))
    accv[...] = acc

    pltpu.sync_copy(accv, shared.at[sid])
    plsc.subcore_barrier()

    @pl.when(sid == 0)
    def _():
        pltpu.sync_copy(shared, redv)
        tot = redv[0, :]
        for s_ in range(1, NS):
            tot = tot + redv[s_, :]
        total = jnp.sum(tot) * jnp.float32(1.0 / E)
        outv[...] = jnp.full((L,), total, jnp.float32)
        pltpu.sync_copy(outv, out_hbm.at[cid])


_mesh = plsc.VectorSubcoreMesh(
    core_axis_name="c", subcore_axis_name="s", num_cores=NC, num_subcores=NS)

_sc_call = pl.kernel(
    _body,
    out_type=jax.ShapeDtypeStruct((NC, L), jnp.float32),
    mesh=_mesh,
    scratch_types=[
        pltpu.VMEM((CHUNK,), jnp.int32),
        pltpu.VMEM((CHUNK,), jnp.int32),
        pltpu.VMEM((CHUNK,), jnp.int32),
        pltpu.VMEM((CHUNK,), jnp.int32),
        pltpu.VMEM((CHUNK, D), jnp.float32),
        pltpu.VMEM((CHUNK, D), jnp.float32),
        pltpu.VMEM((CHUNK, D), jnp.float32),
        pltpu.VMEM((CHUNK, D), jnp.float32),
        pltpu.VMEM((GRPS * L * L,), jnp.float32),
        pltpu.VMEM((L,), jnp.float32),
        pltpu.VMEM((NS, L), jnp.float32),
        pltpu.VMEM((L,), jnp.float32),
        pltpu.VMEM_SHARED((NS, L), jnp.float32),
        pltpu.VMEM_SHARED((NZ, D), jnp.float32),
        pltpu.SemaphoreType.DMA,
        pltpu.SemaphoreType.DMA,
        pltpu.SemaphoreType.DMA,
        pltpu.SemaphoreType.DMA,
    ],
    compiler_params=pltpu.CompilerParams(needs_layout_passes=False),
)


@jax.jit
def kernel(z, pos_edge_index, pos_edge_weights, neg_edge_index):
    del pos_edge_weights
    pad = jnp.zeros((NW * PADE - 2 * E,), jnp.int32)
    src = jnp.concatenate(
        [pos_edge_index[0].astype(jnp.int32),
         neg_edge_index[0].astype(jnp.int32), pad])
    dst = jnp.concatenate(
        [pos_edge_index[1].astype(jnp.int32),
         neg_edge_index[1].astype(jnp.int32), pad])
    out = _sc_call(z, src, dst)
    return out[0, 0] + out[1, 0]


# speedup vs baseline: 1.5499x; 1.5499x over previous
import jax
import jax.numpy as jnp
from jax import lax
from jax.experimental import pallas as pl
from jax.experimental.pallas import tpu as pltpu
from jax.experimental.pallas import tpu_sc as plsc

NC = 2
NS = 16
L = 16
NW = NC * NS
D = 128
NZ = 10000
E = 320000
EPW = 2 * E // NW
CHUNK = 80
NCHUNK = EPW // CHUNK
NPAIR = NCHUNK // 2
GRPS = CHUNK // L
KV = D // L
EPS = 1e-15
LN2 = 0.6931471805599453


def _vlog(x):
    bits = plsc.bitcast(x, jnp.int32)
    e = (bits >> 23) - 127
    m = plsc.bitcast((bits & 0x7FFFFF) | 0x3F800000, jnp.float32)
    big = m >= jnp.float32(1.4142135)
    m = jnp.where(big, m * jnp.float32(0.5), m)
    ef = (e + big.astype(jnp.int32)).astype(jnp.float32)
    s = (m - jnp.float32(1.0)) / (m + jnp.float32(1.0))
    u = s * s
    p = jnp.float32(1.0 / 11.0)
    for c in (1.0 / 9.0, 1.0 / 7.0, 1.0 / 5.0, 1.0 / 3.0):
        p = p * u + jnp.float32(c)
    return ef * jnp.float32(LN2) + jnp.float32(2.0) * s * (jnp.float32(1.0) + u * p)


def _body(z_hbm, src_hbm, dst_hbm, out_hbm,
          idx_s0, idx_d0, idx_s1, idx_d1,
          rows_s0, rows_d0, rows_s1, rows_d1,
          scr, accv, redv, outv, shared, z_sh,
          sem_i0, sem_i1, sem_r0, sem_r1):
    cid = lax.axis_index("c")
    sid = lax.axis_index("s")
    wid = sid * NC + cid
    base_w = wid * EPW
    negv = jnp.full((L,), wid >= NW // 2)
    iota = lax.iota(jnp.int32, L)

    @pl.when(sid < NS - 1)
    def _():
        pltpu.sync_copy(z_hbm.at[pl.ds(sid * 624, 624)],
                        z_sh.at[pl.ds(sid * 624, 624)])

    @pl.when(sid == NS - 1)
    def _():
        pltpu.sync_copy(z_hbm.at[pl.ds(9360, 640)],
                        z_sh.at[pl.ds(9360, 640)])

    def fire_idx(c, idx_s, idx_d, sem):
        base = base_w + c * CHUNK
        pltpu.async_copy(src_hbm.at[pl.ds(base, CHUNK)], idx_s, sem)
        pltpu.async_copy(dst_hbm.at[pl.ds(base, CHUNK)], idx_d, sem)

    def drain_idx(idx_s, idx_d, sem):
        pltpu.make_async_copy(src_hbm.at[pl.ds(0, CHUNK)], idx_s, sem).wait()
        pltpu.make_async_copy(src_hbm.at[pl.ds(0, CHUNK)], idx_d, sem).wait()

    def fire_rows(idx_s, idx_d, rows_s, rows_d, sem):
        pltpu.async_copy(z_sh.at[idx_s], rows_s, sem)
        pltpu.async_copy(z_sh.at[idx_d], rows_d, sem)

    def drain_rows(rows_s, rows_d, sem):
        pltpu.make_async_copy(z_hbm.at[pl.ds(0, CHUNK)], rows_s, sem).wait()
        pltpu.make_async_copy(z_hbm.at[pl.ds(0, CHUNK)], rows_d, sem).wait()

    def compute(rows_s, rows_d, acc):
        @plsc.parallel_loop(0, GRPS, carry=acc)
        def grp_body(g, acc_g):
            sbase = g * (L * L)
            for e_ in range(L):
                r = g * L + e_
                a = rows_s[r, pl.ds(0, L)] * rows_d[r, pl.ds(0, L)]
                b = rows_s[r, pl.ds(L, L)] * rows_d[r, pl.ds(L, L)]
                for k in range(2, KV, 2):
                    a = a + rows_s[r, pl.ds(k * L, L)] * rows_d[r, pl.ds(k * L, L)]
                    b = b + rows_s[r, pl.ds((k + 1) * L, L)] * rows_d[r, pl.ds((k + 1) * L, L)]
                scr[pl.ds(sbase + e_ * L, L)] = a + b
            t = plsc.load_gather(scr, [sbase + iota * L])
            for l in range(1, L):
                t = t + plsc.load_gather(scr, [sbase + iota * L + l])
            prob = jnp.float32(1.0) / (jnp.float32(1.0) + jnp.exp(-t))
            arg = jnp.where(negv,
                            (jnp.float32(1.0) - prob) + jnp.float32(EPS),
                            prob + jnp.float32(EPS))
            return acc_g - _vlog(arg)

        return grp_body

    plsc.subcore_barrier()

    pltpu.sync_copy(src_hbm.at[pl.ds(base_w, CHUNK)], idx_s0)
    pltpu.sync_copy(dst_hbm.at[pl.ds(base_w, CHUNK)], idx_d0)
    fire_rows(idx_s0, idx_d0, rows_s0, rows_d0, sem_r0)
    fire_idx(1, idx_s1, idx_d1, sem_i1)

    def pair_body(i, acc):
        drain_idx(idx_s1, idx_d1, sem_i1)
        fire_rows(idx_s1, idx_d1, rows_s1, rows_d1, sem_r1)

        @pl.when(i < NPAIR - 1)
        def _():
            fire_idx(2 * i + 2, idx_s0, idx_d0, sem_i0)

        drain_rows(rows_s0, rows_d0, sem_r0)
        acc = compute(rows_s0, rows_d0, acc)

        @pl.when(i < NPAIR - 1)
        def _():
            drain_idx(idx_s0, idx_d0, sem_i0)
            fire_rows(idx_s0, idx_d0, rows_s0, rows_d0, sem_r0)
            fire_idx(2 * i + 3, idx_s1, idx_d1, sem_i1)

        drain_rows(rows_s1, rows_d1, sem_r1)
        acc = compute(rows_s1, rows_d1, acc)
        return acc

    acc = lax.fori_loop(0, NPAIR, pair_body, jnp.zeros((L,), jnp.float32))
    accv[...] = acc

    pltpu.sync_copy(accv, shared.at[sid])
    plsc.subcore_barrier()

    @pl.when(sid == 0)
    def _():
        pltpu.sync_copy(shared, redv)
        tot = redv[0, :]
        for s_ in range(1, NS):
            tot = tot + redv[s_, :]
        total = jnp.sum(tot) * jnp.float32(1.0 / E)
        outv[...] = jnp.full((L,), total, jnp.float32)
        pltpu.sync_copy(outv, out_hbm.at[cid])


_mesh = plsc.VectorSubcoreMesh(
    core_axis_name="c", subcore_axis_name="s", num_cores=NC, num_subcores=NS)

_sc_call = pl.kernel(
    _body,
    out_type=jax.ShapeDtypeStruct((NC, L), jnp.float32),
    mesh=_mesh,
    scratch_types=[
        pltpu.VMEM((CHUNK,), jnp.int32),
        pltpu.VMEM((CHUNK,), jnp.int32),
        pltpu.VMEM((CHUNK,), jnp.int32),
        pltpu.VMEM((CHUNK,), jnp.int32),
        pltpu.VMEM((CHUNK, D), jnp.float32),
        pltpu.VMEM((CHUNK, D), jnp.float32),
        pltpu.VMEM((CHUNK, D), jnp.float32),
        pltpu.VMEM((CHUNK, D), jnp.float32),
        pltpu.VMEM((GRPS * L * L,), jnp.float32),
        pltpu.VMEM((L,), jnp.float32),
        pltpu.VMEM((NS, L), jnp.float32),
        pltpu.VMEM((L,), jnp.float32),
        pltpu.VMEM_SHARED((NS, L), jnp.float32),
        pltpu.VMEM_SHARED((NZ, D), jnp.float32),
        pltpu.SemaphoreType.DMA,
        pltpu.SemaphoreType.DMA,
        pltpu.SemaphoreType.DMA,
        pltpu.SemaphoreType.DMA,
    ],
    compiler_params=pltpu.CompilerParams(needs_layout_passes=False),
)


@jax.jit
def kernel(z, pos_edge_index, pos_edge_weights, neg_edge_index):
    del pos_edge_weights
    src = jnp.concatenate(
        [pos_edge_index[0], neg_edge_index[0]]).astype(jnp.int32)
    dst = jnp.concatenate(
        [pos_edge_index[1], neg_edge_index[1]]).astype(jnp.int32)
    out = _sc_call(z, src, dst)
    return out[0, 0] + out[1, 0]


# speedup vs baseline: 1.9206x; 1.2392x over previous
import jax
import jax.numpy as jnp
from jax import lax
from jax.experimental import pallas as pl
from jax.experimental.pallas import tpu as pltpu
from jax.experimental.pallas import tpu_sc as plsc

NC = 2
NS = 16
L = 16
NW = NC * NS
D = 128
NZ = 10000
E = 320000
EPW = 2 * E // NW
CHUNK = 80
NCHUNK = EPW // CHUNK
NPAIR = NCHUNK // 2
GRPS = CHUNK // L
KV = D // L
EPS = 1e-15
LN2 = 0.6931471805599453


def _vlog(x):
    bits = plsc.bitcast(x, jnp.int32)
    e = (bits >> 23) - 127
    m = plsc.bitcast((bits & 0x7FFFFF) | 0x3F800000, jnp.float32)
    big = m >= jnp.float32(1.4142135)
    m = jnp.where(big, m * jnp.float32(0.5), m)
    ef = (e + big.astype(jnp.int32)).astype(jnp.float32)
    s = (m - jnp.float32(1.0)) / (m + jnp.float32(1.0))
    u = s * s
    p = jnp.float32(1.0 / 11.0)
    for c in (1.0 / 9.0, 1.0 / 7.0, 1.0 / 5.0, 1.0 / 3.0):
        p = p * u + jnp.float32(c)
    return ef * jnp.float32(LN2) + jnp.float32(2.0) * s * (jnp.float32(1.0) + u * p)


def _body(z_hbm, src_hbm, dst_hbm, out_hbm,
          idx_s0, idx_d0, idx_s1, idx_d1,
          rows_s0, rows_d0, rows_s1, rows_d1,
          scr, accv, redv, outv, shared, z_sh,
          sem_i0, sem_i1, sem_r0, sem_r1):
    cid = lax.axis_index("c")
    sid = lax.axis_index("s")
    wid = sid * NC + cid
    base_w = wid * EPW
    negv = jnp.full((L,), wid >= NW // 2)
    iota = lax.iota(jnp.int32, L)

    @pl.when(sid < NS - 1)
    def _():
        pltpu.sync_copy(z_hbm.at[pl.ds(sid * 624, 624)],
                        z_sh.at[pl.ds(sid * 624, 624)])

    @pl.when(sid == NS - 1)
    def _():
        pltpu.sync_copy(z_hbm.at[pl.ds(9360, 640)],
                        z_sh.at[pl.ds(9360, 640)])

    def fire_idx(c, idx_s, idx_d, sem):
        base = base_w + c * CHUNK
        pltpu.async_copy(src_hbm.at[pl.ds(base, CHUNK)], idx_s, sem)
        pltpu.async_copy(dst_hbm.at[pl.ds(base, CHUNK)], idx_d, sem)

    def drain_idx(idx_s, idx_d, sem):
        pltpu.make_async_copy(src_hbm.at[pl.ds(0, CHUNK)], idx_s, sem).wait()
        pltpu.make_async_copy(src_hbm.at[pl.ds(0, CHUNK)], idx_d, sem).wait()

    def fire_rows(idx_s, idx_d, rows_s, rows_d, sem):
        pltpu.async_copy(z_sh.at[idx_s], rows_s, sem)
        pltpu.async_copy(z_sh.at[idx_d], rows_d, sem)

    def drain_rows(rows_s, rows_d, sem):
        pltpu.make_async_copy(z_hbm.at[pl.ds(0, CHUNK)], rows_s, sem).wait()
        pltpu.make_async_copy(z_hbm.at[pl.ds(0, CHUNK)], rows_d, sem).wait()

    def compute(rows_s, rows_d, acc):
        def grp_body(g, acc_g):
            for e_ in range(L):
                r = g * L + e_
                a = rows_s[r, pl.ds(0, L)] * rows_d[r, pl.ds(0, L)]
                for k in range(1, KV):
                    a = a + rows_s[r, pl.ds(k * L, L)] * rows_d[r, pl.ds(k * L, L)]
                scr[pl.ds(e_ * L, L)] = a
            t = plsc.load_gather(scr, [iota * L])
            for l in range(1, L):
                t = t + plsc.load_gather(scr, [iota * L + l])
            prob = jnp.float32(1.0) / (jnp.float32(1.0) + jnp.exp(-t))
            arg = jnp.where(negv,
                            (jnp.float32(1.0) - prob) + jnp.float32(EPS),
                            prob + jnp.float32(EPS))
            return acc_g - _vlog(arg)

        return lax.fori_loop(0, GRPS, grp_body, acc)

    plsc.subcore_barrier()

    pltpu.sync_copy(src_hbm.at[pl.ds(base_w, CHUNK)], idx_s0)
    pltpu.sync_copy(dst_hbm.at[pl.ds(base_w, CHUNK)], idx_d0)
    fire_rows(idx_s0, idx_d0, rows_s0, rows_d0, sem_r0)
    fire_idx(1, idx_s1, idx_d1, sem_i1)

    def pair_body(i, acc):
        drain_idx(idx_s1, idx_d1, sem_i1)
        fire_rows(idx_s1, idx_d1, rows_s1, rows_d1, sem_r1)

        @pl.when(i < NPAIR - 1)
        def _():
            fire_idx(2 * i + 2, idx_s0, idx_d0, sem_i0)

        drain_rows(rows_s0, rows_d0, sem_r0)
        acc = compute(rows_s0, rows_d0, acc)

        @pl.when(i < NPAIR - 1)
        def _():
            drain_idx(idx_s0, idx_d0, sem_i0)
            fire_rows(idx_s0, idx_d0, rows_s0, rows_d0, sem_r0)
            fire_idx(2 * i + 3, idx_s1, idx_d1, sem_i1)

        drain_rows(rows_s1, rows_d1, sem_r1)
        acc = compute(rows_s1, rows_d1, acc)
        return acc

    acc = lax.fori_loop(0, NPAIR, pair_body, jnp.zeros((L,), jnp.float32))
    accv[...] = acc

    pltpu.sync_copy(accv, shared.at[sid])
    plsc.subcore_barrier()

    @pl.when(sid == 0)
    def _():
        pltpu.sync_copy(shared, redv)
        tot = redv[0, :]
        for s_ in range(1, NS):
            tot = tot + redv[s_, :]
        total = jnp.sum(tot) * jnp.float32(1.0 / E)
        outv[...] = jnp.full((L,), total, jnp.float32)
        pltpu.sync_copy(outv, out_hbm.at[cid])


_mesh = plsc.VectorSubcoreMesh(
    core_axis_name="c", subcore_axis_name="s", num_cores=NC, num_subcores=NS)

_sc_call = pl.kernel(
    _body,
    out_type=jax.ShapeDtypeStruct((NC, L), jnp.float32),
    mesh=_mesh,
    scratch_types=[
        pltpu.VMEM((CHUNK,), jnp.int32),
        pltpu.VMEM((CHUNK,), jnp.int32),
        pltpu.VMEM((CHUNK,), jnp.int32),
        pltpu.VMEM((CHUNK,), jnp.int32),
        pltpu.VMEM((CHUNK, D), jnp.float32),
        pltpu.VMEM((CHUNK, D), jnp.float32),
        pltpu.VMEM((CHUNK, D), jnp.float32),
        pltpu.VMEM((CHUNK, D), jnp.float32),
        pltpu.VMEM((L * L,), jnp.float32),
        pltpu.VMEM((L,), jnp.float32),
        pltpu.VMEM((NS, L), jnp.float32),
        pltpu.VMEM((L,), jnp.float32),
        pltpu.VMEM_SHARED((NS, L), jnp.float32),
        pltpu.VMEM_SHARED((NZ, D), jnp.float32),
        pltpu.SemaphoreType.DMA,
        pltpu.SemaphoreType.DMA,
        pltpu.SemaphoreType.DMA,
        pltpu.SemaphoreType.DMA,
    ],
    compiler_params=pltpu.CompilerParams(needs_layout_passes=False),
)


@jax.jit
def kernel(z, pos_edge_index, pos_edge_weights, neg_edge_index):
    del pos_edge_weights
    src = jnp.concatenate(
        [pos_edge_index[0], neg_edge_index[0]]).astype(jnp.int32)
    dst = jnp.concatenate(
        [pos_edge_index[1], neg_edge_index[1]]).astype(jnp.int32)
    out = _sc_call(z, src, dst)
    return out[0, 0] + out[1, 0]


# speedup vs baseline: 1.9569x; 1.0189x over previous
import jax
import jax.numpy as jnp
from jax import lax
from jax.experimental import pallas as pl
from jax.experimental.pallas import tpu as pltpu
from jax.experimental.pallas import tpu_sc as plsc

NC = 2
NS = 16
L = 16
NW = NC * NS
D = 128
NZ = 10000
E = 320000
EPW = 2 * E // NW
CHUNK = 80
NCHUNK = EPW // CHUNK
NPAIR = NCHUNK // 2
GRPS = CHUNK // L
KV = D // L
EPS = 1e-15
LN2 = 0.6931471805599453


def _vlog(x):
    bits = plsc.bitcast(x, jnp.int32)
    e = (bits >> 23) - 127
    m = plsc.bitcast((bits & 0x7FFFFF) | 0x3F800000, jnp.float32)
    big = m >= jnp.float32(1.4142135)
    m = jnp.where(big, m * jnp.float32(0.5), m)
    ef = (e + big.astype(jnp.int32)).astype(jnp.float32)
    s = (m - jnp.float32(1.0)) / (m + jnp.float32(1.0))
    u = s * s
    p = jnp.float32(1.0 / 11.0)
    for c in (1.0 / 9.0, 1.0 / 7.0, 1.0 / 5.0, 1.0 / 3.0):
        p = p * u + jnp.float32(c)
    return ef * jnp.float32(LN2) + jnp.float32(2.0) * s * (jnp.float32(1.0) + u * p)


def _body(z_hbm, src_hbm, dst_hbm, out_hbm,
          idx_s0, idx_d0, idx_s1, idx_d1,
          rows_s0, rows_d0, rows_s1, rows_d1,
          scr, accv, redv, outv, shared, z_sh,
          sem_i0, sem_i1, sem_r0, sem_r1):
    cid = lax.axis_index("c")
    sid = lax.axis_index("s")
    wid = sid * NC + cid
    base_w = wid * EPW
    negv = jnp.full((L,), wid >= NW // 2)
    iota = lax.iota(jnp.int32, L)

    @pl.when(sid < NS - 1)
    def _():
        pltpu.sync_copy(z_hbm.at[pl.ds(sid * 624, 624)],
                        z_sh.at[pl.ds(sid * 624, 624)])

    @pl.when(sid == NS - 1)
    def _():
        pltpu.sync_copy(z_hbm.at[pl.ds(9360, 640)],
                        z_sh.at[pl.ds(9360, 640)])

    def fire_idx(c, idx_s, idx_d, sem):
        base = base_w + c * CHUNK
        pltpu.async_copy(src_hbm.at[pl.ds(base, CHUNK)], idx_s, sem)
        pltpu.async_copy(dst_hbm.at[pl.ds(base, CHUNK)], idx_d, sem)

    def drain_idx(idx_s, idx_d, sem):
        pltpu.make_async_copy(src_hbm.at[pl.ds(0, CHUNK)], idx_s, sem).wait()
        pltpu.make_async_copy(src_hbm.at[pl.ds(0, CHUNK)], idx_d, sem).wait()

    def fire_rows(idx_s, idx_d, rows_s, rows_d, sem):
        pltpu.async_copy(z_sh.at[idx_s], rows_s, sem)
        pltpu.async_copy(z_sh.at[idx_d], rows_d, sem)

    def drain_rows(rows_s, rows_d, sem):
        pltpu.make_async_copy(z_hbm.at[pl.ds(0, CHUNK)], rows_s, sem).wait()
        pltpu.make_async_copy(z_hbm.at[pl.ds(0, CHUNK)], rows_d, sem).wait()

    def compute(rows_s, rows_d, acc):
        def grp_body(g, acc_g):
            for e_ in range(L):
                r = g * L + e_
                a = rows_s[r, pl.ds(0, L)] * rows_d[r, pl.ds(0, L)]
                for k in range(1, KV):
                    a = a + rows_s[r, pl.ds(k * L, L)] * rows_d[r, pl.ds(k * L, L)]
                scr[pl.ds(e_ * L, L)] = a
            base = iota * L
            t0 = plsc.load_gather(scr, [base])
            t1 = plsc.load_gather(scr, [base + 1])
            t2 = plsc.load_gather(scr, [base + 2])
            t3 = plsc.load_gather(scr, [base + 3])
            for l in range(4, L, 4):
                t0 = t0 + plsc.load_gather(scr, [base + l])
                t1 = t1 + plsc.load_gather(scr, [base + (l + 1)])
                t2 = t2 + plsc.load_gather(scr, [base + (l + 2)])
                t3 = t3 + plsc.load_gather(scr, [base + (l + 3)])
            t = (t0 + t1) + (t2 + t3)
            prob = jnp.float32(1.0) / (jnp.float32(1.0) + jnp.exp(-t))
            arg = jnp.where(negv,
                            (jnp.float32(1.0) - prob) + jnp.float32(EPS),
                            prob + jnp.float32(EPS))
            return acc_g - _vlog(arg)

        return lax.fori_loop(0, GRPS, grp_body, acc)

    plsc.subcore_barrier()

    pltpu.sync_copy(src_hbm.at[pl.ds(base_w, CHUNK)], idx_s0)
    pltpu.sync_copy(dst_hbm.at[pl.ds(base_w, CHUNK)], idx_d0)
    fire_rows(idx_s0, idx_d0, rows_s0, rows_d0, sem_r0)
    fire_idx(1, idx_s1, idx_d1, sem_i1)

    def pair_body(i, acc):
        drain_idx(idx_s1, idx_d1, sem_i1)
        fire_rows(idx_s1, idx_d1, rows_s1, rows_d1, sem_r1)

        @pl.when(i < NPAIR - 1)
        def _():
            fire_idx(2 * i + 2, idx_s0, idx_d0, sem_i0)

        drain_rows(rows_s0, rows_d0, sem_r0)
        acc = compute(rows_s0, rows_d0, acc)

        @pl.when(i < NPAIR - 1)
        def _():
            drain_idx(idx_s0, idx_d0, sem_i0)
            fire_rows(idx_s0, idx_d0, rows_s0, rows_d0, sem_r0)
            fire_idx(2 * i + 3, idx_s1, idx_d1, sem_i1)

        drain_rows(rows_s1, rows_d1, sem_r1)
        acc = compute(rows_s1, rows_d1, acc)
        return acc

    acc = lax.fori_loop(0, NPAIR, pair_body, jnp.zeros((L,), jnp.float32))
    accv[...] = acc

    pltpu.sync_copy(accv, shared.at[sid])
    plsc.subcore_barrier()

    @pl.when(sid == 0)
    def _():
        pltpu.sync_copy(shared, redv)
        tot = redv[0, :]
        for s_ in range(1, NS):
            tot = tot + redv[s_, :]
        total = jnp.sum(tot) * jnp.float32(1.0 / E)
        outv[...] = jnp.full((L,), total, jnp.float32)
        pltpu.sync_copy(outv, out_hbm.at[cid])


_mesh = plsc.VectorSubcoreMesh(
    core_axis_name="c", subcore_axis_name="s", num_cores=NC, num_subcores=NS)

_sc_call = pl.kernel(
    _body,
    out_type=jax.ShapeDtypeStruct((NC, L), jnp.float32),
    mesh=_mesh,
    scratch_types=[
        pltpu.VMEM((CHUNK,), jnp.int32),
        pltpu.VMEM((CHUNK,), jnp.int32),
        pltpu.VMEM((CHUNK,), jnp.int32),
        pltpu.VMEM((CHUNK,), jnp.int32),
        pltpu.VMEM((CHUNK, D), jnp.float32),
        pltpu.VMEM((CHUNK, D), jnp.float32),
        pltpu.VMEM((CHUNK, D), jnp.float32),
        pltpu.VMEM((CHUNK, D), jnp.float32),
        pltpu.VMEM((L * L,), jnp.float32),
        pltpu.VMEM((L,), jnp.float32),
        pltpu.VMEM((NS, L), jnp.float32),
        pltpu.VMEM((L,), jnp.float32),
        pltpu.VMEM_SHARED((NS, L), jnp.float32),
        pltpu.VMEM_SHARED((NZ, D), jnp.float32),
        pltpu.SemaphoreType.DMA,
        pltpu.SemaphoreType.DMA,
        pltpu.SemaphoreType.DMA,
        pltpu.SemaphoreType.DMA,
    ],
    compiler_params=pltpu.CompilerParams(needs_layout_passes=False),
)


@jax.jit
def kernel(z, pos_edge_index, pos_edge_weights, neg_edge_index):
    del pos_edge_weights
    src = jnp.concatenate(
        [pos_edge_index[0], neg_edge_index[0]]).astype(jnp.int32)
    dst = jnp.concatenate(
        [pos_edge_index[1], neg_edge_index[1]]).astype(jnp.int32)
    out = _sc_call(z, src, dst)
    return out[0, 0] + out[1, 0]
